# Initial kernel scaffold; baseline (speedup 1.0000x reference)
import jax
import jax.numpy as jnp
from jax.experimental import pallas as pl


def kernel(idx, targets, logits_table):
    raise NotImplementedError("write your pallas kernel here")


# speedup vs baseline: 1.2263x; 1.2263x over previous
import functools

import jax
import jax.numpy as jnp
from jax import lax
from jax.experimental import pallas as pl
from jax.experimental.pallas import tpu as pltpu
from jax.experimental.pallas import tpu_sc as plsc

V = 1000
B = 16384
NC = 2
NS = 16
NW = NC * NS
BPW = B // NW
CH = 64
NCHUNK = BPW // CH


def _adj_body(t_ref, o_ref):
    t = t_ref[...]
    m = jnp.max(t, axis=1, keepdims=True)
    lse = m + jnp.log(jnp.sum(jnp.exp(t - m), axis=1, keepdims=True))
    o_ref[...] = t - lse


_adj_call = pl.pallas_call(
    _adj_body,
    out_shape=jax.ShapeDtypeStruct((V, V), jnp.float32),
)


_sc_mesh = plsc.VectorSubcoreMesh(core_axis_name="c", subcore_axis_name="s")


@functools.partial(
    pl.kernel,
    out_type=(
        jax.ShapeDtypeStruct((B, V), jnp.float32),
        jax.ShapeDtypeStruct((NW, 16), jnp.float32),
    ),
    mesh=_sc_mesh,
    compiler_params=pltpu.CompilerParams(use_tc_tiling_on_sc=False),
    scratch_types=[
        pltpu.VMEM((BPW,), jnp.int32),
        pltpu.VMEM((BPW,), jnp.int32),
        pltpu.VMEM((BPW,), jnp.int32),
        pltpu.VMEM((BPW,), jnp.float32),
        pltpu.VMEM((CH, V), jnp.float32),
        pltpu.VMEM((16,), jnp.float32),
        pltpu.SemaphoreType.DMA,
        pltpu.SemaphoreType.DMA,
    ],
)
def _sc_gather(table_hbm, adjflat_hbm, idx_hbm, tgt_hbm, out_hbm,
               part_hbm, idx_v, tgt_v, fidx_v, picked_v, rows_v,
               acc_v, sem, sem2):
    wid = lax.axis_index("s") * NC + lax.axis_index("c")
    base = wid * BPW
    pltpu.sync_copy(idx_hbm.at[pl.ds(base, BPW)], idx_v)
    pltpu.sync_copy(tgt_hbm.at[pl.ds(base, BPW)], tgt_v)
    for i in range(BPW // 16):
        s = pl.ds(i * 16, 16)
        fidx_v[s] = idx_v[s] * V + tgt_v[s]
    pick_cp = pltpu.async_copy(adjflat_hbm.at[fidx_v], picked_v, sem2)
    for k in range(NCHUNK):
        pltpu.async_copy(table_hbm.at[idx_v.at[pl.ds(k * CH, CH)]],
                         rows_v, sem).wait()
        pltpu.sync_copy(rows_v, out_hbm.at[pl.ds(base + k * CH, CH)])
    pick_cp.wait()
    acc = jnp.zeros((16,), jnp.float32)
    for i in range(BPW // 16):
        acc = acc + picked_v[pl.ds(i * 16, 16)]
    acc_v[...] = acc
    pltpu.sync_copy(acc_v, part_hbm.at[wid])


def _loss_body(p_ref, o_ref):
    o_ref[...] = (-jnp.sum(p_ref[...]) / B).reshape(1, 1)


_loss_call = pl.pallas_call(
    _loss_body,
    out_shape=jax.ShapeDtypeStruct((1, 1), jnp.float32),
)


def kernel(idx, targets, logits_table):
    idx32 = idx.astype(jnp.int32)
    tgt32 = targets.astype(jnp.int32)
    adjflat = lax.optimization_barrier(_adj_call(logits_table).reshape(V * V))
    logits, parts = _sc_gather(logits_table, adjflat, idx32, tgt32)
    loss = _loss_call(parts)[0, 0]
    return logits, loss
